# baseline (speedup 1.0000x reference)
import functools

import jax
import jax.numpy as jnp
from jax import lax
from jax.experimental import pallas as pl
from jax.experimental.pallas import tpu as pltpu
from jax.experimental.pallas import tpu_sc as plsc


def _sc_route_scatter(batch, classes, num_classes):
    info = plsc.get_sparse_core_info()
    nc, ns = info.num_cores, info.num_subcores
    nw = nc * ns
    b, d = batch.shape
    bw = b // nw
    nch = b // 16
    own_ch = bw // 16
    mesh = plsc.VectorSubcoreMesh(core_axis_name="c", subcore_axis_name="s")

    @functools.partial(
        pl.kernel,
        mesh=mesh,
        out_type=(
            jax.ShapeDtypeStruct((b, d), batch.dtype),
            jax.ShapeDtypeStruct((b,), jnp.int32),
            jax.ShapeDtypeStruct((16,), jnp.int32),
        ),
        scratch_types=[
            pltpu.VMEM((b,), jnp.int32),
            pltpu.VMEM((bw,), jnp.int32),
            pltpu.VMEM((bw, d), batch.dtype),
            pltpu.VMEM((16,), jnp.int32),
            pltpu.VMEM((16 * num_classes,), jnp.int32),
            pltpu.VMEM((16 * num_classes,), jnp.int32),
            pltpu.SemaphoreType.DMA,
        ],
    )
    def k(batch_hbm, cls_hbm, sx_hbm, pos_hbm, offs_hbm,
          cls_v, pos_v, rows_v, offs_v, tot_v, cnt_v, sem):
        w = lax.axis_index("s") * nc + lax.axis_index("c")
        base = w * bw
        pltpu.sync_copy(cls_hbm, cls_v)
        lane = lax.iota(jnp.int32, 16)
        zeros = jnp.zeros((16,), jnp.int32)
        ones = zeros + 1

        def eqmask(v, e):
            return ones - jnp.minimum(jnp.abs(v - (zeros + e)), ones)

        def scan_incl(x):
            for kk in (1, 2, 4, 8):
                idx = jnp.maximum(lane - kk, zeros)
                sh = x.at[idx].get(mode="promise_in_bounds")
                gate = jnp.minimum(jnp.maximum(lane - (kk - 1), zeros), ones)
                x = x + sh * gate
            return x

        def splat_last(x):
            return x.at[zeros + 15].get(mode="promise_in_bounds")

        for ec in range(num_classes):
            tot_v[pl.ds(ec * 16, 16)] = zeros
            cnt_v[pl.ds(ec * 16, 16)] = zeros

        def count_into(acc_ref):
            def body(i, carry):
                v = cls_v[pl.ds(i * 16, 16)]
                for ec in range(num_classes):
                    acc_ref[pl.ds(ec * 16, 16)] = (
                        acc_ref[pl.ds(ec * 16, 16)] + eqmask(v, ec)
                    )
                return carry

            return body

        lax.fori_loop(0, nch, count_into(tot_v), jnp.int32(0))
        lax.fori_loop(0, w * own_ch, count_into(cnt_v), jnp.int32(0))

        tot_sp = [
            splat_last(scan_incl(tot_v[pl.ds(ec * 16, 16)]))
            for ec in range(num_classes)
        ]
        run_sp = [
            splat_last(scan_incl(cnt_v[pl.ds(ec * 16, 16)]))
            for ec in range(num_classes)
        ]
        offs_sp = [zeros]
        for ec in range(num_classes):
            offs_sp.append(offs_sp[ec] + tot_sp[ec])

        for j in range(own_ch):
            v = cls_v[pl.ds(base + j * 16, 16)]
            pc = zeros
            for ec in range(num_classes):
                mi = eqmask(v, ec)
                inc = scan_incl(mi)
                pc = pc + mi * (offs_sp[ec] + run_sp[ec] + inc - ones)
                run_sp[ec] = run_sp[ec] + splat_last(inc)
            pos_v[pl.ds(j * 16, 16)] = pc

        pltpu.sync_copy(pos_v, pos_hbm.at[pl.ds(base, bw)])

        @pl.when(w == 0)
        def _():
            offs_vec = zeros
            for ec in range(num_classes + 1):
                offs_vec = offs_vec + eqmask(lane, ec) * offs_sp[ec]
            offs_v[...] = offs_vec
            pltpu.sync_copy(offs_v, offs_hbm)

        pltpu.sync_copy(batch_hbm.at[pl.ds(base, bw)], rows_v)
        pltpu.async_copy(rows_v, sx_hbm.at[pos_v], sem).wait()

    return k(batch, classes)


def _fused_body(
    n_tiles, tile_m, num_experts, out_size, b,
    offs_ref, x_ref, w_ref, pos_ref, out_ref, smm_ref, acc_ref,
):
    t = pl.program_id(0)

    @pl.when(t < n_tiles)
    def _mm():
        r0 = t * tile_m
        for e in range(num_experts):
            seg_lo = offs_ref[e]
            seg_hi = offs_ref[e + 1]

            @pl.when((seg_lo < r0 + tile_m) & (seg_hi > r0))
            def _():
                sub = lax.dot_general(
                    x_ref[...],
                    w_ref[e],
                    (((1,), (1,)), ((), ())),
                    precision=lax.Precision.DEFAULT,
                    preferred_element_type=jnp.float32,
                )
                rows = r0 + lax.broadcasted_iota(jnp.int32, (tile_m, out_size), 0)
                m = (rows >= seg_lo) & (rows < seg_hi)
                acc_ref[...] = jnp.where(m, sub, acc_ref[...])

        smm_ref[pl.ds(r0, tile_m), :] = acc_ref[...].astype(jnp.bfloat16)

    @pl.when(t >= n_tiles)
    def _unpermute():
        prow = pos_ref[0].astype(jnp.float32)
        eye = (
            lax.broadcasted_iota(jnp.int32, (tile_m, tile_m), 0)
            == lax.broadcasted_iota(jnp.int32, (tile_m, tile_m), 1)
        ).astype(jnp.float32)
        pcol = lax.dot_general(
            eye, prow, (((1,), (1,)), ((), ())),
            preferred_element_type=jnp.float32,
        )
        slots = lax.broadcasted_iota(jnp.int32, (tile_m, b), 1).astype(jnp.float32)
        onehot = (slots == pcol).astype(jnp.bfloat16)
        out_ref[...] = lax.dot_general(
            onehot, smm_ref[...], (((1,), (0,)), ((), ())),
            preferred_element_type=jnp.float32,
        )


def kernel(batch, classes, W):
    b, in_size = batch.shape
    e, out_size, _ = W.shape
    clz = classes.astype(jnp.int32)

    sorted_x, pos, offs = _sc_route_scatter(batch, clz, e)

    tile_m = 256
    n_tiles = b // tile_m
    pos_3d = pos.reshape(n_tiles, 1, tile_m)
    body = functools.partial(_fused_body, n_tiles, tile_m, e, out_size, b)
    return pl.pallas_call(
        body,
        grid=(2 * n_tiles,),
        in_specs=[
            pl.BlockSpec(memory_space=pltpu.SMEM),
            pl.BlockSpec(
                (tile_m, in_size), lambda t: (jnp.minimum(t, n_tiles - 1), 0)
            ),
            pl.BlockSpec((e, out_size, in_size), lambda t: (0, 0, 0)),
            pl.BlockSpec(
                (1, 1, tile_m), lambda t: (jnp.maximum(t - n_tiles, 0), 0, 0)
            ),
        ],
        out_specs=pl.BlockSpec(
            (tile_m, out_size), lambda t: (jnp.maximum(t - n_tiles, 0), 0)
        ),
        out_shape=jax.ShapeDtypeStruct((b, out_size), jnp.float32),
        scratch_shapes=[
            pltpu.VMEM((b, out_size), jnp.bfloat16),
            pltpu.VMEM((tile_m, out_size), jnp.float32),
        ],
    )(offs, sorted_x, W, pos_3d)


# speedup vs baseline: 1.0891x; 1.0891x over previous
import functools

import jax
import jax.numpy as jnp
from jax import lax
from jax.experimental import pallas as pl
from jax.experimental.pallas import tpu as pltpu
from jax.experimental.pallas import tpu_sc as plsc


def _sc_route_scatter(batch, classes, num_classes):
    info = plsc.get_sparse_core_info()
    nc, ns = info.num_cores, info.num_subcores
    nw = nc * ns
    b, d = batch.shape
    bw = b // nw
    nch = b // 16
    own_ch = bw // 16
    mesh = plsc.VectorSubcoreMesh(core_axis_name="c", subcore_axis_name="s")

    @functools.partial(
        pl.kernel,
        mesh=mesh,
        out_type=(
            jax.ShapeDtypeStruct((b, d), batch.dtype),
            jax.ShapeDtypeStruct((b,), jnp.int32),
            jax.ShapeDtypeStruct((16,), jnp.int32),
        ),
        scratch_types=[
            pltpu.VMEM((b,), jnp.int32),
            pltpu.VMEM((bw,), jnp.int32),
            pltpu.VMEM((bw, d), batch.dtype),
            pltpu.VMEM((16,), jnp.int32),
            pltpu.SemaphoreType.DMA,
            pltpu.SemaphoreType.DMA,
            pltpu.SemaphoreType.DMA,
        ],
    )
    def k(batch_hbm, cls_hbm, sx_hbm, pos_hbm, offs_hbm,
          cls_v, pos_v, rows_v, offs_v, sem, sem2, sem3):
        w = lax.axis_index("s") * nc + lax.axis_index("c")
        base = w * bw
        rows_cp = pltpu.async_copy(batch_hbm.at[pl.ds(base, bw)], rows_v, sem)
        pltpu.sync_copy(cls_hbm, cls_v)
        lane = lax.iota(jnp.int32, 16)
        zeros = jnp.zeros((16,), jnp.int32)
        ones = zeros + 1

        def eqmask(v, e):
            return ones - jnp.minimum(jnp.abs(v - (zeros + e)), ones)

        def scan_incl(x):
            for kk in (1, 2, 4, 8):
                idx = jnp.maximum(lane - kk, zeros)
                sh = x.at[idx].get(mode="promise_in_bounds")
                gate = jnp.minimum(jnp.maximum(lane - (kk - 1), zeros), ones)
                x = x + sh * gate
            return x

        def splat_last(x):
            return x.at[zeros + 15].get(mode="promise_in_bounds")

        first_ch = lax.broadcast_in_dim(w * own_ch, (16,), ())

        def body(i, carry):
            tots, cnts = carry
            v = cls_v[pl.ds(i * 16, 16)]
            iv = lax.broadcast_in_dim(i, (16,), ())
            before = jnp.minimum(jnp.maximum(first_ch - iv, zeros), ones)
            new_t, new_c = [], []
            for ec in range(num_classes):
                mi = eqmask(v, ec)
                new_t.append(tots[ec] + mi)
                new_c.append(cnts[ec] + mi * before)
            return tuple(new_t), tuple(new_c)

        tots, cnts = lax.fori_loop(
            0, nch, body,
            ((zeros,) * num_classes, (zeros,) * num_classes),
        )

        tot_sp = [splat_last(scan_incl(tots[ec])) for ec in range(num_classes)]
        run_sp = [splat_last(scan_incl(cnts[ec])) for ec in range(num_classes)]
        offs_sp = [zeros]
        for ec in range(num_classes):
            offs_sp.append(offs_sp[ec] + tot_sp[ec])

        for j in range(own_ch):
            v = cls_v[pl.ds(base + j * 16, 16)]
            pc = zeros
            for ec in range(num_classes):
                mi = eqmask(v, ec)
                inc = scan_incl(mi)
                pc = pc + mi * (offs_sp[ec] + run_sp[ec] + inc - ones)
                run_sp[ec] = run_sp[ec] + splat_last(inc)
            pos_v[pl.ds(j * 16, 16)] = pc

        pos_cp = pltpu.async_copy(pos_v, pos_hbm.at[pl.ds(base, bw)], sem2)

        @pl.when(w == 0)
        def _():
            offs_vec = zeros
            for ec in range(num_classes + 1):
                offs_vec = offs_vec + eqmask(lane, ec) * offs_sp[ec]
            offs_v[...] = offs_vec
            pltpu.sync_copy(offs_v, offs_hbm)

        rows_cp.wait()
        pltpu.async_copy(rows_v, sx_hbm.at[pos_v], sem3).wait()
        pos_cp.wait()

    return k(batch, classes)


def _fused_body(
    n_tiles, tile_m, num_experts, out_size, b,
    offs_ref, x_ref, w_ref, pos_ref, out_ref, smm_ref, acc_ref,
):
    t = pl.program_id(0)

    @pl.when(t < n_tiles)
    def _mm():
        r0 = t * tile_m
        for e in range(num_experts):
            seg_lo = offs_ref[e]
            seg_hi = offs_ref[e + 1]

            @pl.when((seg_lo < r0 + tile_m) & (seg_hi > r0))
            def _():
                sub = lax.dot_general(
                    x_ref[...],
                    w_ref[e],
                    (((1,), (1,)), ((), ())),
                    precision=lax.Precision.DEFAULT,
                    preferred_element_type=jnp.float32,
                )
                rows = r0 + lax.broadcasted_iota(jnp.int32, (tile_m, out_size), 0)
                m = (rows >= seg_lo) & (rows < seg_hi)
                acc_ref[...] = jnp.where(m, sub, acc_ref[...])

        smm_ref[pl.ds(r0, tile_m), :] = acc_ref[...].astype(jnp.bfloat16)

    @pl.when(t >= n_tiles)
    def _unpermute():
        prow = pos_ref[0].astype(jnp.float32)
        eye = (
            lax.broadcasted_iota(jnp.int32, (tile_m, tile_m), 0)
            == lax.broadcasted_iota(jnp.int32, (tile_m, tile_m), 1)
        ).astype(jnp.float32)
        pcol = lax.dot_general(
            eye, prow, (((1,), (1,)), ((), ())),
            preferred_element_type=jnp.float32,
        )
        slots = lax.broadcasted_iota(jnp.int32, (tile_m, b), 1).astype(jnp.float32)
        onehot = (slots == pcol).astype(jnp.bfloat16)
        out_ref[...] = lax.dot_general(
            onehot, smm_ref[...], (((1,), (0,)), ((), ())),
            preferred_element_type=jnp.float32,
        )


def kernel(batch, classes, W):
    b, in_size = batch.shape
    e, out_size, _ = W.shape
    clz = classes.astype(jnp.int32)

    sorted_x, pos, offs = _sc_route_scatter(batch, clz, e)

    tile_m = 256
    n_tiles = b // tile_m
    pos_3d = pos.reshape(n_tiles, 1, tile_m)
    body = functools.partial(_fused_body, n_tiles, tile_m, e, out_size, b)
    return pl.pallas_call(
        body,
        grid=(2 * n_tiles,),
        in_specs=[
            pl.BlockSpec(memory_space=pltpu.SMEM),
            pl.BlockSpec(
                (tile_m, in_size), lambda t: (jnp.minimum(t, n_tiles - 1), 0)
            ),
            pl.BlockSpec((e, out_size, in_size), lambda t: (0, 0, 0)),
            pl.BlockSpec(
                (1, 1, tile_m), lambda t: (jnp.maximum(t - n_tiles, 0), 0, 0)
            ),
        ],
        out_specs=pl.BlockSpec(
            (tile_m, out_size), lambda t: (jnp.maximum(t - n_tiles, 0), 0)
        ),
        out_shape=jax.ShapeDtypeStruct((b, out_size), jnp.float32),
        scratch_shapes=[
            pltpu.VMEM((b, out_size), jnp.bfloat16),
            pltpu.VMEM((tile_m, out_size), jnp.float32),
        ],
    )(offs, sorted_x, W, pos_3d)
